# Initial kernel scaffold; baseline (speedup 1.0000x reference)
import jax
import jax.numpy as jnp
from jax.experimental import pallas as pl


def kernel(triplets, node_emb, W):
    raise NotImplementedError("write your pallas kernel here")


# speedup vs baseline: 3.4546x; 3.4546x over previous
import functools

import jax
import jax.numpy as jnp
from jax import lax
from jax.experimental import pallas as pl
from jax.experimental.pallas import tpu as pltpu
from jax.experimental.pallas import tpu_sc as plsc

_N = 320000
_D = 128
_NC = 2
_NS = 16
_NW = _NC * _NS
_PER_W = _N // _NW
_CHUNK = 80
_NCHUNK = _PER_W // _CHUNK
_G = _CHUNK // 16


def _body(left_hbm, mid_hbm, right_hbm, node_hbm, w_hbm, out_hbm,
          lidx, midx, ridx, lbuf, wbuf, rbuf, score_v, sem):
    wid = lax.axis_index("s") * _NC + lax.axis_index("c")
    base_w = wid * _PER_W

    def chunk_step(ci, carry):
        base = base_w + ci * _CHUNK
        pltpu.sync_copy(left_hbm.at[pl.ds(base, _CHUNK)], lidx)
        pltpu.sync_copy(mid_hbm.at[pl.ds(base, _CHUNK)], midx)
        pltpu.sync_copy(right_hbm.at[pl.ds(base, _CHUNK)], ridx)
        cl = pltpu.async_copy(node_hbm.at[lidx], lbuf, sem)
        cw = pltpu.async_copy(w_hbm.at[midx], wbuf, sem)
        cr = pltpu.async_copy(node_hbm.at[ridx], rbuf, sem)
        cl.wait()
        cw.wait()
        cr.wait()
        lane15 = lax.iota(jnp.int32, 16) == 15

        def t_step(t, carry):
            acc = jnp.zeros((16,), jnp.float32)
            for k in range(_D // 16):
                l = lbuf[t, pl.ds(k * 16, 16)]
                w = wbuf[t, pl.ds(k * 16, 16)]
                r = rbuf[t, pl.ds(k * 16, 16)]
                acc = acc + l * w * r
            cs = jnp.cumsum(acc)
            idxv = jnp.full((16,), t, jnp.int32)
            plsc.store_scatter(score_v, [idxv], cs, mask=lane15)
            return carry

        lax.fori_loop(0, _CHUNK, t_step, 0, unroll=4)
        pltpu.sync_copy(score_v, out_hbm.at[pl.ds(base, _CHUNK)])
        return carry

    lax.fori_loop(0, _NCHUNK, chunk_step, 0)


@functools.partial(jax.jit, static_argnums=())
def _run(left, mid, right, node_emb, w):
    mesh = plsc.VectorSubcoreMesh(core_axis_name="c", subcore_axis_name="s")
    kfn = pl.kernel(
        _body,
        out_type=jax.ShapeDtypeStruct((_N,), jnp.float32),
        mesh=mesh,
        compiler_params=pltpu.CompilerParams(needs_layout_passes=False),
        scratch_types=[
            pltpu.VMEM((_CHUNK,), jnp.int32),
            pltpu.VMEM((_CHUNK,), jnp.int32),
            pltpu.VMEM((_CHUNK,), jnp.int32),
            pltpu.VMEM((_CHUNK, _D), jnp.float32),
            pltpu.VMEM((_CHUNK, _D), jnp.float32),
            pltpu.VMEM((_CHUNK, _D), jnp.float32),
            pltpu.VMEM((_CHUNK,), jnp.float32),
            pltpu.SemaphoreType.DMA,
        ],
    )
    return kfn(left, mid, right, node_emb, w)


def kernel(triplets, node_emb, W):
    t = triplets.astype(jnp.int32)
    return _run(t[:, 0], t[:, 1], t[:, 2], node_emb, W)


# speedup vs baseline: 7.4593x; 2.1593x over previous
import functools

import jax
import jax.numpy as jnp
from jax import lax
from jax.experimental import pallas as pl
from jax.experimental.pallas import tpu as pltpu
from jax.experimental.pallas import tpu_sc as plsc

_N = 320000
_D = 128
_NNODE = 10000
_NC = 2
_NS = 16
_NW = _NC * _NS
_PER_W = _N // _NW
_CHUNK = 80
_NCHUNK = _PER_W // _CHUNK


def _body(idx_hbm, table_hbm, out_hbm,
          idx_all, lb0, wb0, rb0, lb1, wb1, rb1, sc0, sc1,
          gsem0, gsem1, ssem0, ssem1):
    wid = lax.axis_index("s") * _NC + lax.axis_index("c")
    cbase = wid * _NCHUNK
    tbase = wid * _PER_W
    bufs = ((lb0, wb0, rb0, sc0, gsem0, ssem0),
            (lb1, wb1, rb1, sc1, gsem1, ssem1))
    lane15 = lax.iota(jnp.int32, 16) == 15

    pltpu.sync_copy(idx_hbm.at[pl.ds(cbase, _NCHUNK)], idx_all)

    def fire_gather(ci, s):
        lb, wb, rb, _, gsem, _ = bufs[s]
        pltpu.make_async_copy(table_hbm.at[idx_all.at[ci, 0]], lb, gsem).start()
        pltpu.make_async_copy(table_hbm.at[idx_all.at[ci, 1]], wb, gsem).start()
        pltpu.make_async_copy(table_hbm.at[idx_all.at[ci, 2]], rb, gsem).start()

    def wait_gather(ci, s):
        lb, wb, rb, _, gsem, _ = bufs[s]
        pltpu.make_async_copy(table_hbm.at[idx_all.at[ci, 0]], lb, gsem).wait()
        pltpu.make_async_copy(table_hbm.at[idx_all.at[ci, 1]], wb, gsem).wait()
        pltpu.make_async_copy(table_hbm.at[idx_all.at[ci, 2]], rb, gsem).wait()

    def fire_store(ci, s):
        scv, ssem = bufs[s][3], bufs[s][5]
        dst = out_hbm.at[pl.ds(tbase + ci * _CHUNK, _CHUNK)]
        pltpu.make_async_copy(scv, dst, ssem).start()

    def wait_store(s):
        scv, ssem = bufs[s][3], bufs[s][5]
        pltpu.make_async_copy(scv, out_hbm.at[pl.ds(tbase, _CHUNK)], ssem).wait()

    def compute(ci, s):
        lb, wb, rb, scv = bufs[s][0], bufs[s][1], bufs[s][2], bufs[s][3]

        def t_step(t, carry):
            acc = jnp.zeros((16,), jnp.float32)
            for k in range(_D // 16):
                l = lb[t, pl.ds(k * 16, 16)]
                w = wb[t, pl.ds(k * 16, 16)]
                r = rb[t, pl.ds(k * 16, 16)]
                acc = acc + l * w * r
            cs = jnp.cumsum(acc)
            plsc.store_scatter(scv, [jnp.full((16,), t, jnp.int32)], cs,
                               mask=lane15)
            return carry

        lax.fori_loop(0, _CHUNK, t_step, 0, unroll=4)

    fire_gather(0, 0)

    def pair_step(k, carry):
        for s in (0, 1):
            i = 2 * k + s
            wait_gather(i, s)
            fire_gather(i + 1, 1 - s)
            @pl.when(k >= 1)
            def _():
                wait_store(s)
            compute(i, s)
            fire_store(i, s)
        return carry

    lax.fori_loop(0, (_NCHUNK - 1) // 2, pair_step, 0)

    wait_gather(_NCHUNK - 1, 0)
    wait_store(0)
    compute(_NCHUNK - 1, 0)
    fire_store(_NCHUNK - 1, 0)
    wait_store(1)
    wait_store(0)


@jax.jit
def _run(idx3, table):
    mesh = plsc.VectorSubcoreMesh(core_axis_name="c", subcore_axis_name="s")
    kfn = pl.kernel(
        _body,
        out_type=jax.ShapeDtypeStruct((_N,), jnp.float32),
        mesh=mesh,
        compiler_params=pltpu.CompilerParams(needs_layout_passes=False),
        scratch_types=[
            pltpu.VMEM((_NCHUNK, 3, _CHUNK), jnp.int32),
            pltpu.VMEM((_CHUNK, _D), jnp.float32),
            pltpu.VMEM((_CHUNK, _D), jnp.float32),
            pltpu.VMEM((_CHUNK, _D), jnp.float32),
            pltpu.VMEM((_CHUNK, _D), jnp.float32),
            pltpu.VMEM((_CHUNK, _D), jnp.float32),
            pltpu.VMEM((_CHUNK, _D), jnp.float32),
            pltpu.VMEM((_CHUNK,), jnp.float32),
            pltpu.VMEM((_CHUNK,), jnp.float32),
            pltpu.SemaphoreType.DMA,
            pltpu.SemaphoreType.DMA,
            pltpu.SemaphoreType.DMA,
            pltpu.SemaphoreType.DMA,
        ],
    )
    return kfn(idx3, table)


def kernel(triplets, node_emb, W):
    t = triplets.astype(jnp.int32)
    li = t[:, 0].reshape(-1, _CHUNK)
    mi = (t[:, 1] + _NNODE).reshape(-1, _CHUNK)
    ri = t[:, 2].reshape(-1, _CHUNK)
    idx3 = jnp.stack([li, mi, ri], axis=1)
    table = jnp.concatenate([node_emb, W], axis=0)
    return _run(idx3, table)
